# Initial kernel scaffold; baseline (speedup 1.0000x reference)
import jax
import jax.numpy as jnp
from jax.experimental import pallas as pl


def kernel(x, edge_attr, edge_index, batch_ind, W_embed, b_embed, W_msg, b_msg, W_upd, b_upd, W_act, b_act, W_node, b_node, W_val, b_val):
    raise NotImplementedError("write your pallas kernel here")


# speedup vs baseline: 1.1081x; 1.1081x over previous
import functools

import jax
import jax.numpy as jnp
from jax.experimental import pallas as pl
from jax.experimental.pallas import tpu as pltpu

N = 50000
E = 800000
F = 3
DE = 4
D = 128
T = 3
G = 16

NB = 2048
NBLK = 25
NP = NB * NBLK


def _leaky(v):
    return jnp.maximum(v, 0.0) + 0.01 * jnp.minimum(v, 0.0)


def _b16(v):
    return v.astype(jnp.bfloat16).astype(jnp.float32)


def _embed_prep_body(x_ref, we_ref, be_ref, w1_ref, w2_ref, bm_ref,
                     h_ref, a_ref, b_ref):
    xb = _b16(x_ref[...])
    we = _b16(we_ref[...])
    h = (xb[:, 0:1] * we[0:1, :] + xb[:, 1:2] * we[1:2, :]
         + xb[:, 2:3] * we[2:3, :] + be_ref[...])
    h = _leaky(h)
    h_ref[...] = h
    h16 = h.astype(jnp.bfloat16)
    a_ref[...] = jnp.dot(h16, w1_ref[...].astype(jnp.bfloat16), preferred_element_type=jnp.float32) + bm_ref[...]
    b_ref[...] = jnp.dot(h16, w2_ref[...].astype(jnp.bfloat16), preferred_element_type=jnp.float32)


def _embed_prep(xp, We, be, W1, W2, bm):
    return pl.pallas_call(
        _embed_prep_body,
        grid=(NBLK,),
        in_specs=[
            pl.BlockSpec((NB, F), lambda i: (i, 0)),
            pl.BlockSpec((F, D), lambda i: (0, 0)),
            pl.BlockSpec((1, D), lambda i: (0, 0)),
            pl.BlockSpec((D, D), lambda i: (0, 0)),
            pl.BlockSpec((D, D), lambda i: (0, 0)),
            pl.BlockSpec((1, D), lambda i: (0, 0)),
        ],
        out_specs=[pl.BlockSpec((NB, D), lambda i: (i, 0))] * 3,
        out_shape=[jax.ShapeDtypeStruct((NP, D), jnp.float32)] * 3,
    )(xp, We, be, W1, W2, bm)


def _update_prep_body(h_ref, g_ref, wu1_ref, wu2_ref, bu_ref,
                      w1_ref, w2_ref, bm_ref, hn_ref, a_ref, b_ref):
    i = pl.program_id(0)
    row = i * NB + jax.lax.broadcasted_iota(jnp.int32, (NB, 1), 0)
    g = jnp.where(row < N, g_ref[...], 0.0)
    hn = _leaky(jnp.dot(h_ref[...].astype(jnp.bfloat16), wu1_ref[...].astype(jnp.bfloat16), preferred_element_type=jnp.float32)
                + jnp.dot(g.astype(jnp.bfloat16), wu2_ref[...].astype(jnp.bfloat16), preferred_element_type=jnp.float32)
                + bu_ref[...])
    hn_ref[...] = hn
    hn16 = hn.astype(jnp.bfloat16)
    a_ref[...] = jnp.dot(hn16, w1_ref[...].astype(jnp.bfloat16), preferred_element_type=jnp.float32) + bm_ref[...]
    b_ref[...] = jnp.dot(hn16, w2_ref[...].astype(jnp.bfloat16), preferred_element_type=jnp.float32)


def _update_prep(h, agg, Wu1, Wu2, bu, W1, W2, bm):
    return pl.pallas_call(
        _update_prep_body,
        grid=(NBLK,),
        in_specs=[
            pl.BlockSpec((NB, D), lambda i: (i, 0)),
            pl.BlockSpec((NB, D), lambda i: (i, 0)),
            pl.BlockSpec((D, D), lambda i: (0, 0)),
            pl.BlockSpec((D, D), lambda i: (0, 0)),
            pl.BlockSpec((1, D), lambda i: (0, 0)),
            pl.BlockSpec((D, D), lambda i: (0, 0)),
            pl.BlockSpec((D, D), lambda i: (0, 0)),
            pl.BlockSpec((1, D), lambda i: (0, 0)),
        ],
        out_specs=[pl.BlockSpec((NB, D), lambda i: (i, 0))] * 3,
        out_shape=[jax.ShapeDtypeStruct((NP, D), jnp.float32)] * 3,
    )(h, agg, Wu1, Wu2, bu, W1, W2, bm)


def _edge_c_body(ea_ref, w3_ref, c_ref):
    ea = _b16(ea_ref[...])
    w3 = _b16(w3_ref[...])
    c = (ea[:, 0:1] * w3[0:1, :] + ea[:, 1:2] * w3[1:2, :]
         + ea[:, 2:3] * w3[2:3, :] + ea[:, 3:4] * w3[3:4, :])
    c_ref[...] = c


EB = 8000
EBLK = E // EB


def _edge_c(ea, W3):
    return pl.pallas_call(
        _edge_c_body,
        grid=(EBLK,),
        in_specs=[
            pl.BlockSpec((EB, DE), lambda i: (i, 0)),
            pl.BlockSpec((DE, D), lambda i: (0, 0)),
        ],
        out_specs=pl.BlockSpec((EB, D), lambda i: (i, 0)),
        out_shape=jax.ShapeDtypeStruct((E, D), jnp.float32),
    )(ea, W3)


def _onehot(bi):
    return jnp.where(bi == jax.lax.broadcasted_iota(jnp.int32, (NB, G), 1),
                     1.0, 0.0).astype(jnp.float32)


def _heads1_body(h_ref, bi_ref, wn_ref, bn_ref,
                 logits_ref, segh_ref, cnt_ref, smax_ref):
    i = pl.program_id(0)
    row = i * NB + jax.lax.broadcasted_iota(jnp.int32, (NB, 1), 0)
    valid = row < N
    hb = jnp.where(valid, h_ref[...], 0.0)
    oh = _onehot(bi_ref[...])
    logits = jnp.dot(hb.astype(jnp.bfloat16), wn_ref[...].astype(jnp.bfloat16), preferred_element_type=jnp.float32) + bn_ref[...]
    logits_ref[...] = logits

    @pl.when(i == 0)
    def _():
        segh_ref[...] = jnp.zeros_like(segh_ref)
        cnt_ref[...] = jnp.zeros_like(cnt_ref)
        smax_ref[...] = jnp.full_like(smax_ref, -1e30)

    segh_ref[...] += jax.lax.dot_general(oh, hb, (((0,), (0,)), ((), ())),
                                         preferred_element_type=jnp.float32, precision=jax.lax.Precision.HIGHEST)
    cnt_ref[...] += jax.lax.dot_general(oh, jnp.ones_like(hb), (((0,), (0,)), ((), ())),
                                        preferred_element_type=jnp.float32, precision=jax.lax.Precision.HIGHEST)
    for j in range(4):
        mj = jnp.max(jnp.where(oh > 0, logits[:, j:j + 1], -1e30),
                     axis=0, keepdims=True)
        smax_ref[j:j + 1, :] = jnp.maximum(smax_ref[j:j + 1, :], mj)


def _heads1(h, bi, wn, bn):
    return pl.pallas_call(
        _heads1_body,
        grid=(NBLK,),
        in_specs=[
            pl.BlockSpec((NB, D), lambda i: (i, 0)),
            pl.BlockSpec((NB, 1), lambda i: (i, 0)),
            pl.BlockSpec((D, 4), lambda i: (0, 0)),
            pl.BlockSpec((1, 4), lambda i: (0, 0)),
        ],
        out_specs=[
            pl.BlockSpec((NB, 4), lambda i: (i, 0)),
            pl.BlockSpec((G, D), lambda i: (0, 0)),
            pl.BlockSpec((G, D), lambda i: (0, 0)),
            pl.BlockSpec((4, G), lambda i: (0, 0)),
        ],
        out_shape=[
            jax.ShapeDtypeStruct((NP, 4), jnp.float32),
            jax.ShapeDtypeStruct((G, D), jnp.float32),
            jax.ShapeDtypeStruct((G, D), jnp.float32),
            jax.ShapeDtypeStruct((4, G), jnp.float32),
        ],
    )(h, bi, wn, bn)


def _heads2a_body(logits_ref, bi_ref, smax_ref, segh_ref, cnt_ref,
                  wact_ref, bact_ref, wval_ref, bval_ref,
                  denom_ref, act_ref, val_ref):
    i = pl.program_id(0)
    oh = _onehot(bi_ref[...])
    sel_max = jax.lax.dot_general(oh, smax_ref[...], (((1,), (1,)), ((), ())),
                                  preferred_element_type=jnp.float32, precision=jax.lax.Precision.HIGHEST)
    ex = jnp.exp(logits_ref[...] - sel_max) * (oh.sum(axis=1, keepdims=True))

    @pl.when(i == 0)
    def _():
        denom_ref[...] = jnp.zeros_like(denom_ref)

    denom_ref[...] += jax.lax.dot_general(oh, ex, (((0,), (0,)), ((), ())),
                                          preferred_element_type=jnp.float32, precision=jax.lax.Precision.HIGHEST)

    @pl.when(i == NBLK - 1)
    def _():
        pooled = segh_ref[...] / jnp.maximum(cnt_ref[...], 1.0)
        p16 = pooled.astype(jnp.bfloat16)
        val_ref[...] = jnp.dot(p16, wval_ref[...].astype(jnp.bfloat16),
                               preferred_element_type=jnp.float32) + bval_ref[...]
        al = jnp.dot(p16, wact_ref[...].astype(jnp.bfloat16),
                     preferred_element_type=jnp.float32) + bact_ref[...]
        am = jnp.max(al, axis=1, keepdims=True)
        e = jnp.exp(al - am)
        act_ref[...] = e / jnp.sum(e, axis=1, keepdims=True)


def _heads2b_body(logits_ref, bi_ref, smax_ref, denom_ref, ns_ref):
    oh = _onehot(bi_ref[...])
    sel_max = jax.lax.dot_general(oh, smax_ref[...], (((1,), (1,)), ((), ())),
                                  preferred_element_type=jnp.float32, precision=jax.lax.Precision.HIGHEST)
    ex = jnp.exp(logits_ref[...] - sel_max)
    den = jnp.dot(oh, denom_ref[...], preferred_element_type=jnp.float32, precision=jax.lax.Precision.HIGHEST)
    ns_ref[...] = ex / den


def kernel(x, edge_attr, edge_index, batch_ind, W_embed, b_embed, W_msg, b_msg,
           W_upd, b_upd, W_act, b_act, W_node, b_node, W_val, b_val):
    src = edge_index[0]
    dst = edge_index[1]
    xp = jnp.pad(x, ((0, NP - N), (0, 0)))
    bi = jnp.pad(batch_ind.astype(jnp.int32), (0, NP - N),
                 constant_values=G).reshape(NP, 1)

    Wm1 = W_msg[:, :D, :]
    Wm2 = W_msg[:, D:2 * D, :]
    Wm3 = W_msg[:, 2 * D:, :]
    Wu1 = W_upd[:, :D, :]
    Wu2 = W_upd[:, D:, :]

    h, A, B = _embed_prep(xp, W_embed, b_embed.reshape(1, D),
                          Wm1[0], Wm2[0], b_msg[0].reshape(1, D))
    for t in range(T):
        C = _edge_c(edge_attr, Wm3[t])
        msg = _leaky(jnp.take(A, src, axis=0) + jnp.take(B, dst, axis=0) + C)
        agg = jax.ops.segment_sum(msg, dst, num_segments=NP)
        t2 = min(t + 1, T - 1)
        h, A, B = _update_prep(h, agg, Wu1[t], Wu2[t], b_upd[t].reshape(1, D),
                               Wm1[t2], Wm2[t2], b_msg[t2].reshape(1, D))

    logits, segh, cnt, smax = _heads1(h, bi, W_node, b_node.reshape(1, 4))

    denom, act, val = pl.pallas_call(
        _heads2a_body,
        grid=(NBLK,),
        in_specs=[
            pl.BlockSpec((NB, 4), lambda i: (i, 0)),
            pl.BlockSpec((NB, 1), lambda i: (i, 0)),
            pl.BlockSpec((4, G), lambda i: (0, 0)),
            pl.BlockSpec((G, D), lambda i: (0, 0)),
            pl.BlockSpec((G, D), lambda i: (0, 0)),
            pl.BlockSpec((D, 4), lambda i: (0, 0)),
            pl.BlockSpec((1, 4), lambda i: (0, 0)),
            pl.BlockSpec((D, 1), lambda i: (0, 0)),
            pl.BlockSpec((1, 1), lambda i: (0, 0)),
        ],
        out_specs=[
            pl.BlockSpec((G, 4), lambda i: (0, 0)),
            pl.BlockSpec((G, 4), lambda i: (0, 0)),
            pl.BlockSpec((G, 1), lambda i: (0, 0)),
        ],
        out_shape=[
            jax.ShapeDtypeStruct((G, 4), jnp.float32),
            jax.ShapeDtypeStruct((G, 4), jnp.float32),
            jax.ShapeDtypeStruct((G, 1), jnp.float32),
        ],
    )(logits, bi, smax, segh, cnt, W_act, b_act.reshape(1, 4),
      W_val, b_val.reshape(1, 1))

    ns = pl.pallas_call(
        _heads2b_body,
        grid=(NBLK,),
        in_specs=[
            pl.BlockSpec((NB, 4), lambda i: (i, 0)),
            pl.BlockSpec((NB, 1), lambda i: (i, 0)),
            pl.BlockSpec((4, G), lambda i: (0, 0)),
            pl.BlockSpec((G, 4), lambda i: (0, 0)),
        ],
        out_specs=pl.BlockSpec((NB, 4), lambda i: (i, 0)),
        out_shape=jax.ShapeDtypeStruct((NP, 4), jnp.float32),
    )(logits, bi, smax, denom)

    return (act, ns[:N], val)
